# initial kernel scaffold (unmeasured)
import jax
import jax.numpy as jnp
from jax import lax
from jax.experimental import pallas as pl
from jax.experimental.pallas import tpu as pltpu

N_DEV = 4


def kernel(x, w_mat, scale_x, scale_w):
    m_total, k_per = x.shape
    k_total, n = w_mat.shape
    m_per = m_total // N_DEV

    def body(x_ref, w_ref, sx_ref, sw_ref, out_ref,
             x8_ref, recv_ref, send_sems, recv_sems):
        my = lax.axis_index("i")

        barrier_sem = pltpu.get_barrier_semaphore()
        for d in range(1, N_DEV):
            pl.semaphore_signal(
                barrier_sem, inc=1,
                device_id=((my + d) % N_DEV,),
                device_id_type=pl.DeviceIdType.MESH,
            )
        pl.semaphore_wait(barrier_sem, N_DEV - 1)

        x8_ref[...] = x_ref[...].astype(jnp.float8_e4m3fn)

        rdmas = []
        for d in range(1, N_DEV):
            tgt = (my + d) % N_DEV
            rdma = pltpu.make_async_remote_copy(
                src_ref=x8_ref.at[pl.ds(tgt * m_per, m_per), :],
                dst_ref=recv_ref.at[my],
                send_sem=send_sems.at[d - 1],
                recv_sem=recv_sems.at[d - 1],
                device_id=(tgt,),
                device_id_type=pl.DeviceIdType.MESH,
            )
            rdma.start()
            rdmas.append(rdma)

        recv_ref[my] = x8_ref[pl.ds(my * m_per, m_per), :]

        for rdma in rdmas:
            rdma.wait()

        acc = None
        for s in range(N_DEV):
            a = recv_ref[s].astype(jnp.bfloat16)
            b = w_ref[s * k_per:(s + 1) * k_per, :].astype(jnp.bfloat16)
            p = lax.dot_general(
                a, b, (((1,), (0,)), ((), ())),
                preferred_element_type=jnp.float32,
            )
            acc = p if acc is None else acc + p

        y = acc * (sx_ref[0] * sw_ref[0])
        z = jnp.clip(y, -60.0, 60.0)
        out_ref[...] = y * (1.0 / (1.0 + jnp.exp(-z)))

    return pl.pallas_call(
        body,
        out_shape=jax.ShapeDtypeStruct((m_per, n), jnp.float32),
        in_specs=[
            pl.BlockSpec(memory_space=pltpu.VMEM),
            pl.BlockSpec(memory_space=pltpu.VMEM),
            pl.BlockSpec(memory_space=pltpu.SMEM),
            pl.BlockSpec(memory_space=pltpu.SMEM),
        ],
        out_specs=pl.BlockSpec(memory_space=pltpu.VMEM),
        scratch_shapes=[
            pltpu.VMEM((m_total, k_per), jnp.float8_e4m3fn),
            pltpu.VMEM((N_DEV, m_per, k_per), jnp.float8_e4m3fn),
            pltpu.SemaphoreType.DMA((N_DEV - 1,)),
            pltpu.SemaphoreType.DMA((N_DEV - 1,)),
        ],
        compiler_params=pltpu.CompilerParams(collective_id=0),
    )(x, w_mat, scale_x, scale_w)


# baseline (device time: 63741 ns/iter reference)
import jax
import jax.numpy as jnp
from jax import lax
from jax.experimental import pallas as pl
from jax.experimental.pallas import tpu as pltpu

N_DEV = 4


def kernel(x, w_mat, scale_x, scale_w):
    m_total, k_per = x.shape
    k_total, n = w_mat.shape
    m_per = m_total // N_DEV

    def body(x_ref, w_hbm, sx_ref, sw_ref, out_ref,
             x8_ref, recv_ref, w_vmem, send_sems, recv_sems, w_sems):
        my = lax.axis_index("i")

        w_dma0 = pltpu.make_async_copy(
            w_hbm.at[pl.ds(0, k_per), :], w_vmem.at[0], w_sems.at[0])
        w_dma0.start()

        barrier_sem = pltpu.get_barrier_semaphore()
        for d in range(1, N_DEV):
            pl.semaphore_signal(
                barrier_sem, inc=1,
                device_id=((my + d) % N_DEV,),
                device_id_type=pl.DeviceIdType.MESH,
            )
        pl.semaphore_wait(barrier_sem, N_DEV - 1)

        x8_ref[...] = x_ref[...].astype(jnp.float8_e4m3fn)

        rdmas = []
        for d in range(1, N_DEV):
            tgt = (my + d) % N_DEV
            rdma = pltpu.make_async_remote_copy(
                src_ref=x8_ref.at[pl.ds(tgt * m_per, m_per), :],
                dst_ref=recv_ref.at[my],
                send_sem=send_sems.at[d - 1],
                recv_sem=recv_sems.at[d - 1],
                device_id=(tgt,),
                device_id_type=pl.DeviceIdType.MESH,
            )
            rdma.start()
            rdmas.append(rdma)

        recv_ref[my] = x8_ref[pl.ds(my * m_per, m_per), :]

        for rdma in rdmas:
            rdma.wait()

        for s in range(N_DEV):
            if s + 1 < N_DEV:
                nxt = pltpu.make_async_copy(
                    w_hbm.at[pl.ds((s + 1) * k_per, k_per), :],
                    w_vmem.at[(s + 1) % 2], w_sems.at[(s + 1) % 2])
                nxt.start()
            pltpu.make_async_copy(
                w_hbm.at[pl.ds(s * k_per, k_per), :],
                w_vmem.at[s % 2], w_sems.at[s % 2]).wait()
            a = recv_ref[s].astype(jnp.bfloat16)
            b = w_vmem[s % 2].astype(jnp.bfloat16)
            p = lax.dot_general(
                a, b, (((1,), (0,)), ((), ())),
                preferred_element_type=jnp.float32,
            )
            if s == 0:
                out_ref[...] = p
            else:
                out_ref[...] += p

        y = out_ref[...] * (sx_ref[0] * sw_ref[0])
        z = jnp.clip(y, -60.0, 60.0)
        out_ref[...] = y * (1.0 / (1.0 + jnp.exp(-z)))

    return pl.pallas_call(
        body,
        out_shape=jax.ShapeDtypeStruct((m_per, n), jnp.float32),
        in_specs=[
            pl.BlockSpec(memory_space=pltpu.VMEM),
            pl.BlockSpec(memory_space=pltpu.MemorySpace.HBM),
            pl.BlockSpec(memory_space=pltpu.SMEM),
            pl.BlockSpec(memory_space=pltpu.SMEM),
        ],
        out_specs=pl.BlockSpec(memory_space=pltpu.VMEM),
        scratch_shapes=[
            pltpu.VMEM((m_total, k_per), jnp.float8_e4m3fn),
            pltpu.VMEM((N_DEV, m_per, k_per), jnp.float8_e4m3fn),
            pltpu.VMEM((2, k_per, n), jnp.float32),
            pltpu.SemaphoreType.DMA((N_DEV - 1,)),
            pltpu.SemaphoreType.DMA((N_DEV - 1,)),
            pltpu.SemaphoreType.DMA((2,)),
        ],
        compiler_params=pltpu.CompilerParams(
            collective_id=0,
            vmem_limit_bytes=100 * 1024 * 1024,
        ),
    )(x, w_mat, scale_x, scale_w)


# device time: 56350 ns/iter; 1.1312x vs baseline; 1.1312x over previous
import jax
import jax.numpy as jnp
from jax import lax
from jax.experimental import pallas as pl
from jax.experimental.pallas import tpu as pltpu

N_DEV = 4


def kernel(x, w_mat, scale_x, scale_w):
    m_total, k_per = x.shape
    k_total, n = w_mat.shape
    m_per = m_total // N_DEV

    def body(x_hbm, w_hbm, sx_ref, sw_ref, out_ref,
             xs_ref, x8_ref, recv_ref, w_vmem,
             x_sems, w_sems, send_sems, recv_sems):
        my = lax.axis_index("i")

        send_d = (2, 1, 3)
        tgts = [(my + d) % N_DEV for d in send_d]
        comp_srcs = [my, (my - 1) % N_DEV, (my + 1) % N_DEV, (my + 2) % N_DEV]

        def x_dma(j, slot):
            blk = tgts[j] if j < 3 else my
            return pltpu.make_async_copy(
                x_hbm.at[pl.ds(blk * m_per, m_per), :],
                xs_ref.at[slot], x_sems.at[slot])

        def w_dma(i, slot):
            return pltpu.make_async_copy(
                w_hbm.at[pl.ds(comp_srcs[i] * k_per, k_per), :],
                w_vmem.at[slot], w_sems.at[slot])

        w_dma(0, 0).start()
        w_dma(1, 1).start()
        x_dma(0, 0).start()
        x_dma(1, 1).start()

        barrier_sem = pltpu.get_barrier_semaphore()
        for d in range(1, N_DEV):
            pl.semaphore_signal(
                barrier_sem, inc=1,
                device_id=((my + d) % N_DEV,),
                device_id_type=pl.DeviceIdType.MESH,
            )
        pl.semaphore_wait(barrier_sem, N_DEV - 1)

        rdmas = {}
        for j in range(3):
            d = send_d[j]
            x_dma(j, j % 2).wait()
            x8_ref[j] = xs_ref[j % 2].astype(jnp.float8_e4m3fn)
            rdma = pltpu.make_async_remote_copy(
                src_ref=x8_ref.at[j],
                dst_ref=recv_ref.at[my],
                send_sem=send_sems.at[j],
                recv_sem=recv_sems.at[d - 1],
                device_id=(tgts[j],),
                device_id_type=pl.DeviceIdType.MESH,
            )
            rdma.start()
            rdmas[d] = rdma
            x_dma(j + 2, j % 2).start() if j + 2 < 4 else None
        x_dma(3, 1).wait()
        recv_ref[my] = xs_ref[1].astype(jnp.float8_e4m3fn)

        wait_d = (None, 1, 3, 2)
        for i in range(N_DEV):
            if wait_d[i] is not None:
                rdmas[wait_d[i]].wait_recv()
            w_dma(i, i % 2).wait()
            a = recv_ref[comp_srcs[i]].astype(jnp.bfloat16)
            b = w_vmem[i % 2].astype(jnp.bfloat16)
            p = lax.dot_general(
                a, b, (((1,), (0,)), ((), ())),
                preferred_element_type=jnp.float32,
            )
            if i == 0:
                out_ref[...] = p
            else:
                out_ref[...] += p
            if i + 2 < N_DEV:
                w_dma(i + 2, i % 2).start()

        y = out_ref[...] * (sx_ref[0] * sw_ref[0])
        z = jnp.clip(y, -60.0, 60.0)
        out_ref[...] = y * (1.0 / (1.0 + jnp.exp(-z)))

        for d in (2, 1, 3):
            rdmas[d].wait_send()

    return pl.pallas_call(
        body,
        out_shape=jax.ShapeDtypeStruct((m_per, n), jnp.float32),
        in_specs=[
            pl.BlockSpec(memory_space=pltpu.MemorySpace.HBM),
            pl.BlockSpec(memory_space=pltpu.MemorySpace.HBM),
            pl.BlockSpec(memory_space=pltpu.SMEM),
            pl.BlockSpec(memory_space=pltpu.SMEM),
        ],
        out_specs=pl.BlockSpec(memory_space=pltpu.VMEM),
        scratch_shapes=[
            pltpu.VMEM((2, m_per, k_per), jnp.float32),
            pltpu.VMEM((3, m_per, k_per), jnp.float8_e4m3fn),
            pltpu.VMEM((N_DEV, m_per, k_per), jnp.float8_e4m3fn),
            pltpu.VMEM((2, k_per, n), jnp.float32),
            pltpu.SemaphoreType.DMA((2,)),
            pltpu.SemaphoreType.DMA((2,)),
            pltpu.SemaphoreType.DMA((3,)),
            pltpu.SemaphoreType.DMA((3,)),
        ],
        compiler_params=pltpu.CompilerParams(
            collective_id=0,
            vmem_limit_bytes=100 * 1024 * 1024,
        ),
    )(x, w_mat, scale_x, scale_w)


# device time: 52608 ns/iter; 1.2116x vs baseline; 1.0711x over previous
import jax
import jax.numpy as jnp
from jax import lax
from jax.experimental import pallas as pl
from jax.experimental.pallas import tpu as pltpu

N_DEV = 4
N_CH = 2


def kernel(x, w_mat, scale_x, scale_w):
    m_total, k_per = x.shape
    k_total, n = w_mat.shape
    m_per = m_total // N_DEV
    m_ch = m_per // N_CH

    def body(x_hbm, w_hbm, sx_ref, sw_ref, out_ref,
             xs_ref, x8_ref, recv_ref, w_vmem,
             x_sems, w_sems, send_sems, recv_sems):
        my = lax.axis_index("i")

        send_d = (2, 1, 3)
        tgts = [(my + d) % N_DEV for d in send_d]
        comp_srcs = [my, (my - 1) % N_DEV, (my + 1) % N_DEV, (my + 2) % N_DEV]

        def x_dma(c, slot):
            j, h = divmod(c, N_CH)
            blk = tgts[j] if j < 3 else my
            return pltpu.make_async_copy(
                x_hbm.at[pl.ds(blk * m_per + h * m_ch, m_ch), :],
                xs_ref.at[slot], x_sems.at[slot])

        def w_dma(i, slot):
            return pltpu.make_async_copy(
                w_hbm.at[pl.ds(comp_srcs[i] * k_per, k_per), :],
                w_vmem.at[slot], w_sems.at[slot])

        x_dma(0, 0).start()
        x_dma(1, 1).start()
        w_dma(0, 0).start()
        w_dma(1, 1).start()
        w_dma(2, 2).start()

        barrier_sem = pltpu.get_barrier_semaphore()
        for d in range(1, N_DEV):
            pl.semaphore_signal(
                barrier_sem, inc=1,
                device_id=((my + d) % N_DEV,),
                device_id_type=pl.DeviceIdType.MESH,
            )
        pl.semaphore_wait(barrier_sem, N_DEV - 1)

        rdmas = {}
        for c in range(3 * N_CH):
            j, h = divmod(c, N_CH)
            d = send_d[j]
            rows = pl.ds(h * m_ch, m_ch)
            x_dma(c, c % 2).wait()
            x8_ref[j, rows, :] = xs_ref[c % 2].astype(jnp.float8_e4m3fn)
            rdma = pltpu.make_async_remote_copy(
                src_ref=x8_ref.at[j, rows, :],
                dst_ref=recv_ref.at[my, rows, :],
                send_sem=send_sems.at[c],
                recv_sem=recv_sems.at[(d - 1) * N_CH + h],
                device_id=(tgts[j],),
                device_id_type=pl.DeviceIdType.MESH,
            )
            rdma.start()
            rdmas[(d, h)] = rdma
            if c + 2 < 4 * N_CH:
                x_dma(c + 2, c % 2).start()
        for h in range(N_CH):
            c = 3 * N_CH + h
            x_dma(c, c % 2).wait()
            recv_ref[my, pl.ds(h * m_ch, m_ch), :] = (
                xs_ref[c % 2].astype(jnp.float8_e4m3fn))

        sched = [
            (0, 0, None), (0, 1, None),
            (1, 0, 1), (2, 0, 3),
            (1, 1, 1), (2, 1, 3),
            (3, 0, 2), (3, 1, 2),
        ]
        done_w = set()
        for i, h, d in sched:
            if d is not None:
                rdmas[(d, h)].wait_recv()
            if i not in done_w:
                w_dma(i, i % 3).wait()
                done_w.add(i)
            rows = pl.ds(h * m_ch, m_ch)
            a = recv_ref[comp_srcs[i], rows, :].astype(jnp.bfloat16)
            b = w_vmem[i % 3].astype(jnp.bfloat16)
            p = lax.dot_general(
                a, b, (((1,), (0,)), ((), ())),
                preferred_element_type=jnp.float32,
            )
            if i == 0:
                out_ref[rows, :] = p
            else:
                out_ref[rows, :] += p
            if (i, h) == (0, 1):
                w_dma(3, 0).start()

        y = out_ref[...] * (sx_ref[0] * sw_ref[0])
        z = jnp.clip(y, -60.0, 60.0)
        out_ref[...] = y * (1.0 / (1.0 + jnp.exp(-z)))

        for d, h in rdmas:
            rdmas[(d, h)].wait_send()

    return pl.pallas_call(
        body,
        out_shape=jax.ShapeDtypeStruct((m_per, n), jnp.float32),
        in_specs=[
            pl.BlockSpec(memory_space=pltpu.MemorySpace.HBM),
            pl.BlockSpec(memory_space=pltpu.MemorySpace.HBM),
            pl.BlockSpec(memory_space=pltpu.SMEM),
            pl.BlockSpec(memory_space=pltpu.SMEM),
        ],
        out_specs=pl.BlockSpec(memory_space=pltpu.VMEM),
        scratch_shapes=[
            pltpu.VMEM((2, m_ch, k_per), jnp.float32),
            pltpu.VMEM((3, m_per, k_per), jnp.float8_e4m3fn),
            pltpu.VMEM((N_DEV, m_per, k_per), jnp.float8_e4m3fn),
            pltpu.VMEM((3, k_per, n), jnp.float32),
            pltpu.SemaphoreType.DMA((2,)),
            pltpu.SemaphoreType.DMA((3,)),
            pltpu.SemaphoreType.DMA((3 * N_CH,)),
            pltpu.SemaphoreType.DMA((3 * N_CH,)),
        ],
        compiler_params=pltpu.CompilerParams(
            collective_id=0,
            vmem_limit_bytes=110 * 1024 * 1024,
        ),
    )(x, w_mat, scale_x, scale_w)


# device time: 40982 ns/iter; 1.5553x vs baseline; 1.2837x over previous
import jax
import jax.numpy as jnp
from jax import lax
from jax.experimental import pallas as pl
from jax.experimental.pallas import tpu as pltpu

N_DEV = 4
N_CH = 2


def kernel(x, w_mat, scale_x, scale_w):
    m_total, k_per = x.shape
    k_total, n = w_mat.shape
    m_per = m_total // N_DEV
    m_ch = m_per // N_CH

    def body(x_hbm, w_hbm, sx_ref, sw_ref, out_hbm,
             xs_ref, x8_ref, recv_ref, w_vmem, w8_ref, acc_ref,
             x_sems, w_sems, send_sems, recv_sems, out_sems):
        my = lax.axis_index("i")

        send_d = (2, 1, 3)
        tgts = [(my + d) % N_DEV for d in send_d]
        send_seq = [(j, h) for h in range(N_CH) for j in range(3)]
        comp_srcs = [my, (my - 1) % N_DEV, (my + 1) % N_DEV, (my + 2) % N_DEV]

        def x_dma(c, slot):
            if c < 6:
                j, h = send_seq[c]
                blk = tgts[j]
            else:
                h = c - 6
                blk = my
            return pltpu.make_async_copy(
                x_hbm.at[pl.ds(blk * m_per + h * m_ch, m_ch), :],
                xs_ref.at[slot], x_sems.at[slot])

        def w_dma(i, slot):
            return pltpu.make_async_copy(
                w_hbm.at[pl.ds(comp_srcs[i] * k_per, k_per), :],
                w_vmem.at[slot], w_sems.at[slot])

        x_dma(0, 0).start()
        x_dma(1, 1).start()
        w_dma(0, 0).start()
        w_dma(1, 1).start()

        barrier_sem = pltpu.get_barrier_semaphore()
        for d in range(1, N_DEV):
            pl.semaphore_signal(
                barrier_sem, inc=1,
                device_id=((my + d) % N_DEV,),
                device_id_type=pl.DeviceIdType.MESH,
            )
        pl.semaphore_wait(barrier_sem, N_DEV - 1)

        rdmas = {}
        for c in range(6):
            j, h = send_seq[c]
            d = send_d[j]
            rows = pl.ds(h * m_ch, m_ch)
            x_dma(c, c % 2).wait()
            x8_ref[j, rows, :] = xs_ref[c % 2].astype(jnp.float8_e4m3fn)
            rdma = pltpu.make_async_remote_copy(
                src_ref=x8_ref.at[j, rows, :],
                dst_ref=recv_ref.at[my, rows, :],
                send_sem=send_sems.at[c],
                recv_sem=recv_sems.at[(d - 1) * N_CH + h],
                device_id=(tgts[j],),
                device_id_type=pl.DeviceIdType.MESH,
            )
            rdma.start()
            rdmas[(d, h)] = rdma
            x_dma(c + 2, c % 2).start()
        for h in range(N_CH):
            c = 6 + h
            x_dma(c, c % 2).wait()
            recv_ref[my, pl.ds(h * m_ch, m_ch), :] = (
                xs_ref[c % 2].astype(jnp.float8_e4m3fn))

        sched = [
            (0, 0, None), (0, 1, None),
            (1, 0, 1), (2, 0, 3), (3, 0, 2),
            (1, 1, 1), (2, 1, 3), (3, 1, 2),
        ]
        w_ready = set()
        out_dmas = []

        def epilogue_half(h):
            rows = pl.ds(h * m_ch, m_ch)
            y = acc_ref[rows, :] * (sx_ref[0] * sw_ref[0])
            z = jnp.clip(y, -60.0, 60.0)
            acc_ref[rows, :] = y * (1.0 / (1.0 + jnp.exp(-z)))
            odma = pltpu.make_async_copy(
                acc_ref.at[rows, :], out_hbm.at[rows, :], out_sems.at[h])
            odma.start()
            out_dmas.append(odma)

        for i, h, d in sched:
            if d is not None:
                rdmas[(d, h)].wait_recv()
            if i not in w_ready:
                w_dma(i, i % 2).wait()
                w8_ref[i % 3] = w_vmem[i % 2].astype(jnp.float8_e5m2)
                if i + 2 < N_DEV:
                    w_dma(i + 2, i % 2).start()
                w_ready.add(i)
            rows = pl.ds(h * m_ch, m_ch)
            a = recv_ref[comp_srcs[i], rows, :]
            b = w8_ref[i % 3]
            p = lax.dot_general(
                a, b, (((1,), (0,)), ((), ())),
                preferred_element_type=jnp.float32,
            )
            if i == 0:
                acc_ref[rows, :] = p
            else:
                acc_ref[rows, :] += p
            if i == 3:
                epilogue_half(h)

        for odma in out_dmas:
            odma.wait()
        for key in rdmas:
            rdmas[key].wait_send()

    return pl.pallas_call(
        body,
        out_shape=jax.ShapeDtypeStruct((m_per, n), jnp.float32),
        in_specs=[
            pl.BlockSpec(memory_space=pltpu.MemorySpace.HBM),
            pl.BlockSpec(memory_space=pltpu.MemorySpace.HBM),
            pl.BlockSpec(memory_space=pltpu.SMEM),
            pl.BlockSpec(memory_space=pltpu.SMEM),
        ],
        out_specs=pl.BlockSpec(memory_space=pltpu.MemorySpace.HBM),
        scratch_shapes=[
            pltpu.VMEM((2, m_ch, k_per), jnp.float32),
            pltpu.VMEM((3, m_per, k_per), jnp.float8_e4m3fn),
            pltpu.VMEM((N_DEV, m_per, k_per), jnp.float8_e4m3fn),
            pltpu.VMEM((2, k_per, n), jnp.float32),
            pltpu.VMEM((3, k_per, n), jnp.float8_e5m2),
            pltpu.VMEM((m_per, n), jnp.float32),
            pltpu.SemaphoreType.DMA((2,)),
            pltpu.SemaphoreType.DMA((2,)),
            pltpu.SemaphoreType.DMA((6,)),
            pltpu.SemaphoreType.DMA((6,)),
            pltpu.SemaphoreType.DMA((2,)),
        ],
        compiler_params=pltpu.CompilerParams(
            collective_id=0,
            vmem_limit_bytes=110 * 1024 * 1024,
        ),
    )(x, w_mat, scale_x, scale_w)


# device time: 40060 ns/iter; 1.5911x vs baseline; 1.0230x over previous
import jax
import jax.numpy as jnp
from jax import lax
from jax.experimental import pallas as pl
from jax.experimental.pallas import tpu as pltpu

N_DEV = 4
N_CH = 4


def kernel(x, w_mat, scale_x, scale_w):
    m_total, k_per = x.shape
    k_total, n = w_mat.shape
    m_per = m_total // N_DEV
    m_ch = m_per // N_CH

    def body(x_hbm, w_hbm, sx_ref, sw_ref, out_hbm,
             xs_ref, x8_ref, recv_ref, w_vmem, w8_ref, acc_ref,
             x_sems, w_sems, send_sems, recv_sems, out_sems):
        my = lax.axis_index("i")

        send_d = (2, 1, 3)
        tgts = [(my + d) % N_DEV for d in send_d]
        send_seq = [(j, h) for h in range(N_CH) for j in range(3)]
        n_send = 3 * N_CH
        comp_srcs = [my, (my - 1) % N_DEV, (my + 1) % N_DEV, (my + 2) % N_DEV]

        def x_dma(c, slot):
            if c < n_send:
                j, h = send_seq[c]
                blk = tgts[j]
            else:
                h = c - n_send
                blk = my
            return pltpu.make_async_copy(
                x_hbm.at[pl.ds(blk * m_per + h * m_ch, m_ch), :],
                xs_ref.at[slot], x_sems.at[slot])

        def w_dma(i, slot):
            return pltpu.make_async_copy(
                w_hbm.at[pl.ds(comp_srcs[i] * k_per, k_per), :],
                w_vmem.at[slot], w_sems.at[slot])

        x_dma(0, 0).start()
        x_dma(1, 1).start()
        w_dma(0, 0).start()
        w_dma(1, 1).start()

        barrier_sem = pltpu.get_barrier_semaphore()
        for d in range(1, N_DEV):
            pl.semaphore_signal(
                barrier_sem, inc=1,
                device_id=((my + d) % N_DEV,),
                device_id_type=pl.DeviceIdType.MESH,
            )
        pl.semaphore_wait(barrier_sem, N_DEV - 1)

        rdmas = {}
        for c in range(n_send):
            j, h = send_seq[c]
            d = send_d[j]
            rows = pl.ds(h * m_ch, m_ch)
            x_dma(c, c % 2).wait()
            x8_ref[j, rows, :] = xs_ref[c % 2].astype(jnp.float8_e4m3fn)
            rdma = pltpu.make_async_remote_copy(
                src_ref=x8_ref.at[j, rows, :],
                dst_ref=recv_ref.at[my, rows, :],
                send_sem=send_sems.at[c],
                recv_sem=recv_sems.at[(d - 1) * N_CH + h],
                device_id=(tgts[j],),
                device_id_type=pl.DeviceIdType.MESH,
            )
            rdma.start()
            rdmas[(d, h)] = rdma
            x_dma(c + 2, c % 2).start()
        for h in range(N_CH):
            c = n_send + h
            x_dma(c, c % 2).wait()
            recv_ref[my, pl.ds(h * m_ch, m_ch), :] = (
                xs_ref[c % 2].astype(jnp.float8_e4m3fn))
            if c + 2 < n_send + N_CH:
                x_dma(c + 2, c % 2).start()

        sched = [(0, h, None) for h in range(N_CH)] + [
            (i, h, d)
            for h in range(N_CH)
            for i, d in ((1, 1), (2, 3), (3, 2))
        ]
        w_ready = set()
        out_dmas = []

        def epilogue_half(h):
            rows = pl.ds(h * m_ch, m_ch)
            y = acc_ref[rows, :] * (sx_ref[0] * sw_ref[0])
            z = jnp.clip(y, -60.0, 60.0)
            acc_ref[rows, :] = y * (1.0 / (1.0 + jnp.exp(-z)))
            odma = pltpu.make_async_copy(
                acc_ref.at[rows, :], out_hbm.at[rows, :], out_sems.at[h])
            odma.start()
            out_dmas.append(odma)

        for i, h, d in sched:
            if d is not None:
                rdmas[(d, h)].wait_recv()
            if i not in w_ready:
                w_dma(i, i % 2).wait()
                w8_ref[i % 3] = w_vmem[i % 2].astype(jnp.float8_e5m2)
                if i + 2 < N_DEV:
                    w_dma(i + 2, i % 2).start()
                w_ready.add(i)
            rows = pl.ds(h * m_ch, m_ch)
            a = recv_ref[comp_srcs[i], rows, :]
            b = w8_ref[i % 3]
            p = lax.dot_general(
                a, b, (((1,), (0,)), ((), ())),
                preferred_element_type=jnp.float32,
            )
            if i == 0:
                acc_ref[rows, :] = p
            else:
                acc_ref[rows, :] += p
            if i == 3:
                epilogue_half(h)

        for odma in out_dmas:
            odma.wait()
        for key in rdmas:
            rdmas[key].wait_send()

    return pl.pallas_call(
        body,
        out_shape=jax.ShapeDtypeStruct((m_per, n), jnp.float32),
        in_specs=[
            pl.BlockSpec(memory_space=pltpu.MemorySpace.HBM),
            pl.BlockSpec(memory_space=pltpu.MemorySpace.HBM),
            pl.BlockSpec(memory_space=pltpu.SMEM),
            pl.BlockSpec(memory_space=pltpu.SMEM),
        ],
        out_specs=pl.BlockSpec(memory_space=pltpu.MemorySpace.HBM),
        scratch_shapes=[
            pltpu.VMEM((2, m_ch, k_per), jnp.float32),
            pltpu.VMEM((3, m_per, k_per), jnp.float8_e4m3fn),
            pltpu.VMEM((N_DEV, m_per, k_per), jnp.float8_e4m3fn),
            pltpu.VMEM((2, k_per, n), jnp.float32),
            pltpu.VMEM((3, k_per, n), jnp.float8_e5m2),
            pltpu.VMEM((m_per, n), jnp.float32),
            pltpu.SemaphoreType.DMA((2,)),
            pltpu.SemaphoreType.DMA((2,)),
            pltpu.SemaphoreType.DMA((3 * N_CH,)),
            pltpu.SemaphoreType.DMA((3 * N_CH,)),
            pltpu.SemaphoreType.DMA((N_CH,)),
        ],
        compiler_params=pltpu.CompilerParams(
            collective_id=0,
            vmem_limit_bytes=110 * 1024 * 1024,
        ),
    )(x, w_mat, scale_x, scale_w)
